# baseline (speedup 1.0000x reference)
import functools

import jax
import jax.numpy as jnp
from jax import lax
from jax.experimental import pallas as pl
from jax.experimental.pallas import tpu as pltpu
from jax.experimental.pallas import tpu_sc as plsc

_NW = 32
_CHUNK = 128
_LANES = 16
_BB = 512


@functools.lru_cache(maxsize=None)
def _build_sc(batch, nf, vocab, emb):
    b_per_w = batch // _NW
    pairs_per_w = b_per_w * nf
    n_rows_tab = nf * vocab
    halves = b_per_w // _CHUNK
    n_gathers = nf * halves
    n_pairs = n_gathers // 2
    vec_per_b = b_per_w // _LANES

    mesh = plsc.VectorSubcoreMesh(core_axis_name="c", subcore_axis_name="s")

    @functools.partial(
        pl.kernel,
        mesh=mesh,
        compiler_params=pltpu.CompilerParams(needs_layout_passes=False),
        out_type=jax.ShapeDtypeStruct((nf, batch, 2 * emb), jnp.float32),
        scratch_types=[
            pltpu.VMEM((pairs_per_w,), jnp.int32),
            pltpu.VMEM((n_gathers, _CHUNK), jnp.int32),
            pltpu.VMEM((_CHUNK, 2 * emb), jnp.float32),
            pltpu.VMEM((_CHUNK, 2 * emb), jnp.float32),
            pltpu.VMEM_SHARED((n_rows_tab, 2 * emb), jnp.float32),
            pltpu.SemaphoreType.DMA,
            pltpu.SemaphoreType.DMA,
            pltpu.SemaphoreType.DMA,
            pltpu.SemaphoreType.DMA,
        ],
    )
    def k(xflat, tabp, out, xc_v, idx_v, buf0, buf1, tab_sh, g0, g1, s0, s1):
        cid = lax.axis_index("c")
        sid = lax.axis_index("s")
        wid = sid * 2 + cid
        b0 = wid * b_per_w

        @pl.when(sid == 0)
        def _():
            pltpu.sync_copy(tabp, tab_sh)

        pltpu.sync_copy(xflat.at[pl.ds(wid * pairs_per_w, pairs_per_w)], xc_v)

        lane = lax.iota(jnp.int32, _LANES)

        def idx_body(f, _):
            for v in range(vec_per_b):
                xidx = lane * nf + (v * (_LANES * nf) + f)
                xv = plsc.load_gather(xc_v, [xidx])
                row = f * halves + (v * _LANES) // _CHUNK
                col = (v * _LANES) % _CHUNK
                idx_v[row, pl.ds(col, _LANES)] = xv + f * vocab
            return 0

        lax.fori_loop(0, nf, idx_body, 0)

        plsc.subcore_barrier()

        bufs = (buf0, buf1)
        gsems = (g0, g1)
        ssems = (s0, s1)

        def gather(j, b):
            return pltpu.make_async_copy(
                tab_sh.at[idx_v.at[j]], bufs[b], gsems[b]
            )

        def store(j, b):
            f = j // halves
            h = j - f * halves
            return pltpu.make_async_copy(
                bufs[b],
                out.at[f, pl.ds(b0 + h * _CHUNK, _CHUNK), :],
                ssems[b],
            )

        gather(0, 0).start()
        gather(1, 1).start()

        def pair_body(g, _):
            for b in range(2):
                j = g * 2 + b
                gather(j, b).wait()
                store(j, b).start()

                @pl.when(g < n_pairs - 1)
                def _():
                    store(j, b).wait()
                    gather(j + 2, b).start()

            return 0

        lax.fori_loop(0, n_pairs, pair_body, 0)

        store(n_gathers - 2, 0).wait()
        store(n_gathers - 1, 1).wait()

    return k


@functools.lru_cache(maxsize=None)
def _build_tc(batch, nf, emb):
    def body(x_ref, o_ref):
        o_ref[...] = jnp.swapaxes(x_ref[:, :, 0:emb], 1, 2)

    return pl.pallas_call(
        body,
        grid=(nf, batch // _BB),
        in_specs=[pl.BlockSpec((1, _BB, 2 * emb), lambda f, i: (f, i, 0))],
        out_specs=pl.BlockSpec((1, emb, _BB), lambda f, i: (f, 0, i)),
        out_shape=jax.ShapeDtypeStruct((nf, emb, batch), jnp.float32),
    )


def kernel(x_cat, tables):
    batch, nf = x_cat.shape
    _, vocab, emb = tables.shape

    xflat = x_cat.reshape(batch * nf).astype(jnp.int32)
    tabp = jnp.pad(tables.reshape(nf * vocab, emb), ((0, 0), (0, emb)))

    inter = _build_sc(batch, nf, vocab, emb)(xflat, tabp)
    out = _build_tc(batch, nf, emb)(inter)
    return jnp.transpose(out, (2, 0, 1))


# speedup vs baseline: 1.7613x; 1.7613x over previous
import functools

import jax
import jax.numpy as jnp
from jax import lax
from jax.experimental import pallas as pl
from jax.experimental.pallas import tpu as pltpu
from jax.experimental.pallas import tpu_sc as plsc

_NW = 32
_CHUNK = 512
_LANES = 16


@functools.lru_cache(maxsize=None)
def _build(rows_total, n_rows_tab, emb):
    rows_per_w = rows_total // _NW
    n_gathers = rows_per_w // _CHUNK
    n_pairs = n_gathers // 2
    vec_per_gather = _CHUNK // _LANES

    mesh = plsc.VectorSubcoreMesh(core_axis_name="c", subcore_axis_name="s")

    @functools.partial(
        pl.kernel,
        mesh=mesh,
        compiler_params=pltpu.CompilerParams(use_tc_tiling_on_sc=False),
        out_type=jax.ShapeDtypeStruct((rows_total, emb), jnp.float32),
        scratch_types=[
            pltpu.VMEM((rows_per_w,), jnp.int32),
            pltpu.VMEM((rows_per_w,), jnp.int32),
            pltpu.VMEM((n_gathers, _CHUNK), jnp.int32),
            pltpu.VMEM((_CHUNK, emb), jnp.float32),
            pltpu.VMEM((_CHUNK, emb), jnp.float32),
            pltpu.VMEM_SHARED((n_rows_tab, emb), jnp.float32),
            pltpu.SemaphoreType.DMA,
            pltpu.SemaphoreType.DMA,
            pltpu.SemaphoreType.DMA,
            pltpu.SemaphoreType.DMA,
        ],
    )
    def k(xflat, offs, tab, out, xc_v, offs_v, idx_v, buf0, buf1, tab_sh,
          g0, g1, s0, s1):
        cid = lax.axis_index("c")
        sid = lax.axis_index("s")
        wid = sid * 2 + cid
        base = wid * rows_per_w

        @pl.when(sid == 0)
        def _():
            pltpu.sync_copy(tab, tab_sh)

        pltpu.sync_copy(xflat.at[pl.ds(base, rows_per_w)], xc_v)
        pltpu.sync_copy(offs, offs_v)

        def idx_body(j, _):
            for l in range(vec_per_gather):
                fo = j * _CHUNK + l * _LANES
                idx_v[j, pl.ds(l * _LANES, _LANES)] = (
                    xc_v[pl.ds(fo, _LANES)] + offs_v[pl.ds(fo, _LANES)]
                )
            return 0

        lax.fori_loop(0, n_gathers, idx_body, 0)

        plsc.subcore_barrier()

        bufs = (buf0, buf1)
        gsems = (g0, g1)
        ssems = (s0, s1)

        def gather(j, b):
            return pltpu.make_async_copy(
                tab_sh.at[idx_v.at[j]], bufs[b], gsems[b]
            )

        def store(j, b):
            return pltpu.make_async_copy(
                bufs[b],
                out.at[pl.ds(base + j * _CHUNK, _CHUNK)],
                ssems[b],
            )

        gather(0, 0).start()
        gather(1, 1).start()

        def pair_body(g, _):
            for b in range(2):
                j = g * 2 + b
                gather(j, b).wait()
                store(j, b).start()

                @pl.when(g < n_pairs - 1)
                def _():
                    store(j, b).wait()
                    gather(j + 2, b).start()

            return 0

        lax.fori_loop(0, n_pairs, pair_body, 0)

        store(n_gathers - 2, 0).wait()
        store(n_gathers - 1, 1).wait()

    return k


def kernel(x_cat, tables):
    batch, nf = x_cat.shape
    nf2, vocab, emb = tables.shape
    rows_total = batch * nf
    rows_per_w = rows_total // _NW

    xflat = x_cat.reshape(rows_total).astype(jnp.int32)
    tab = tables.reshape(nf2 * vocab, emb)
    offs = jnp.tile(jnp.arange(nf, dtype=jnp.int32) * vocab, rows_per_w // nf)

    out = _build(rows_total, nf2 * vocab, emb)(xflat, offs, tab)
    return out.reshape(batch, nf, emb)
